# initial kernel scaffold (unmeasured)
import jax
import jax.numpy as jnp
from jax import lax
from jax.experimental import pallas as pl
from jax.experimental.pallas import tpu as pltpu

N_DEV = 16
N_STEPS = 4


def _mlp_layer(x_shard, win, wout, collective_id):
    b, _ = x_shard.shape
    _, h_dim = win.shape
    _, d_out = wout.shape

    def body(x_ref, win_ref, wout_ref, out_ref,
             acc_ref, recv_ref, send_sems, recv_sems):
        my_i = lax.axis_index("i")

        acc_ref[:, :] = jnp.dot(
            x_ref[:, :], win_ref[:, :], preferred_element_type=jnp.float32
        )

        for s in range(N_STEPS):
            partner = my_i ^ (1 << s)
            rdma = pltpu.make_async_remote_copy(
                src_ref=acc_ref,
                dst_ref=recv_ref.at[s],
                send_sem=send_sems.at[s],
                recv_sem=recv_sems.at[s],
                device_id=(partner,),
                device_id_type=pl.DeviceIdType.MESH,
            )
            rdma.start()
            rdma.wait()
            acc_ref[:, :] = acc_ref[:, :] + recv_ref[s, :, :]

        h = jnp.maximum(acc_ref[:, :], 0.0)
        out_ref[:, :] = jnp.dot(
            h, wout_ref[:, :], preferred_element_type=jnp.float32
        )

    return pl.pallas_call(
        body,
        out_shape=jax.ShapeDtypeStruct((b, d_out), jnp.float32),
        in_specs=[pl.BlockSpec(memory_space=pltpu.VMEM)] * 3,
        out_specs=pl.BlockSpec(memory_space=pltpu.VMEM),
        scratch_shapes=[
            pltpu.VMEM((b, h_dim), jnp.float32),
            pltpu.VMEM((N_STEPS, b, h_dim), jnp.float32),
            pltpu.SemaphoreType.DMA((N_STEPS,)),
            pltpu.SemaphoreType.DMA((N_STEPS,)),
        ],
        compiler_params=pltpu.CompilerParams(collective_id=collective_id),
    )(x_shard, win, wout)


def kernel(x, Win0, Wout0, Win1, Wout1, Win2, Wout2):
    x = _mlp_layer(x, Win0, Wout0, collective_id=0)
    x = _mlp_layer(x, Win1, Wout1, collective_id=1)
    x = _mlp_layer(x, Win2, Wout2, collective_id=2)
    return x


# baseline (device time: 149013 ns/iter reference)
import jax
import jax.numpy as jnp
from jax import lax
from jax.experimental import pallas as pl
from jax.experimental.pallas import tpu as pltpu

N_DEV = 16
N_STEPS = 4


def _mlp_layer(x_shard, win, wout):
    b, _ = x_shard.shape
    _, h_dim = win.shape
    _, d_out = wout.shape

    def body(x_ref, win_ref, wout_ref, out_ref,
             acc_ref, recv_ref, send_sems, recv_sems):
        my_i = lax.axis_index("i")

        acc_ref[:, :] = jnp.dot(
            x_ref[:, :], win_ref[:, :], preferred_element_type=jnp.float32
        )

        for s in range(N_STEPS):
            partner = my_i ^ (1 << s)
            rdma = pltpu.make_async_remote_copy(
                src_ref=acc_ref,
                dst_ref=recv_ref.at[s],
                send_sem=send_sems.at[s],
                recv_sem=recv_sems.at[s],
                device_id=(partner,),
                device_id_type=pl.DeviceIdType.MESH,
            )
            rdma.start()
            rdma.wait()
            acc_ref[:, :] = acc_ref[:, :] + recv_ref[s, :, :]

        h = jnp.maximum(acc_ref[:, :], 0.0)
        out_ref[:, :] = jnp.dot(
            h, wout_ref[:, :], preferred_element_type=jnp.float32
        )

    return pl.pallas_call(
        body,
        out_shape=jax.ShapeDtypeStruct((b, d_out), jnp.float32),
        in_specs=[pl.BlockSpec(memory_space=pltpu.VMEM)] * 3,
        out_specs=pl.BlockSpec(memory_space=pltpu.VMEM),
        scratch_shapes=[
            pltpu.VMEM((b, h_dim), jnp.float32),
            pltpu.VMEM((N_STEPS, b, h_dim), jnp.float32),
            pltpu.SemaphoreType.DMA((N_STEPS,)),
            pltpu.SemaphoreType.DMA((N_STEPS,)),
        ],
    )(x_shard, win, wout)


def kernel(x, Win0, Wout0, Win1, Wout1, Win2, Wout2):
    x = _mlp_layer(x, Win0, Wout0)
    x = _mlp_layer(x, Win1, Wout1)
    x = _mlp_layer(x, Win2, Wout2)
    return x


# device time: 88682 ns/iter; 1.6803x vs baseline; 1.6803x over previous
import jax
import jax.numpy as jnp
from jax import lax
from jax.experimental import pallas as pl
from jax.experimental.pallas import tpu as pltpu

N_DEV = 16


def _mlp_layer(x_shard, win, wout):
    b, _ = x_shard.shape
    _, h_dim = win.shape
    _, d_out = wout.shape
    chunk = h_dim // N_DEV

    def body(x_ref, win_ref, wout_ref, out_ref,
             acc_ref, rs_recv_ref, hchunk_ref, hfull_ref,
             rs_send_sem, rs_recv_sem, ag_send_sem, ag_recv_sem):
        my_i = lax.axis_index("i")

        acc_ref[:, :] = jnp.dot(
            x_ref[:, :], win_ref[:, :], preferred_element_type=jnp.float32
        )

        for k in range(N_DEV - 1):
            tgt = (my_i + 1 + k) % N_DEV
            rdma = pltpu.make_async_remote_copy(
                src_ref=acc_ref.at[:, pl.ds(tgt * chunk, chunk)],
                dst_ref=rs_recv_ref.at[my_i],
                send_sem=rs_send_sem,
                recv_sem=rs_recv_sem,
                device_id=(tgt,),
                device_id_type=pl.DeviceIdType.MESH,
            )
            rdma.start()

        rs_recv_ref[my_i] = acc_ref[:, pl.ds(my_i * chunk, chunk)]

        for k in range(N_DEV - 1):
            wait = pltpu.make_async_remote_copy(
                src_ref=acc_ref.at[:, pl.ds(0, chunk)],
                dst_ref=rs_recv_ref.at[k],
                send_sem=rs_send_sem,
                recv_sem=rs_recv_sem,
                device_id=(my_i,),
                device_id_type=pl.DeviceIdType.MESH,
            )
            wait.wait_recv()

        h_me = rs_recv_ref[0]
        for j in range(1, N_DEV):
            h_me = h_me + rs_recv_ref[j]
        hchunk_ref[:, :] = jnp.maximum(h_me, 0.0)

        for k in range(N_DEV - 1):
            wait = pltpu.make_async_remote_copy(
                src_ref=acc_ref.at[:, pl.ds(0, chunk)],
                dst_ref=rs_recv_ref.at[k],
                send_sem=rs_send_sem,
                recv_sem=rs_recv_sem,
                device_id=(my_i,),
                device_id_type=pl.DeviceIdType.MESH,
            )
            wait.wait_send()

        for k in range(N_DEV - 1):
            tgt = (my_i + 1 + k) % N_DEV
            rdma = pltpu.make_async_remote_copy(
                src_ref=hchunk_ref,
                dst_ref=hfull_ref.at[:, pl.ds(my_i * chunk, chunk)],
                send_sem=ag_send_sem,
                recv_sem=ag_recv_sem,
                device_id=(tgt,),
                device_id_type=pl.DeviceIdType.MESH,
            )
            rdma.start()

        hfull_ref[:, pl.ds(my_i * chunk, chunk)] = hchunk_ref[:, :]

        for k in range(N_DEV - 1):
            wait = pltpu.make_async_remote_copy(
                src_ref=hchunk_ref,
                dst_ref=hfull_ref.at[:, pl.ds(k * chunk, chunk)],
                send_sem=ag_send_sem,
                recv_sem=ag_recv_sem,
                device_id=(my_i,),
                device_id_type=pl.DeviceIdType.MESH,
            )
            wait.wait_recv()

        out_ref[:, :] = jnp.dot(
            hfull_ref[:, :], wout_ref[:, :], preferred_element_type=jnp.float32
        )

        for k in range(N_DEV - 1):
            wait = pltpu.make_async_remote_copy(
                src_ref=hchunk_ref,
                dst_ref=hfull_ref.at[:, pl.ds(k * chunk, chunk)],
                send_sem=ag_send_sem,
                recv_sem=ag_recv_sem,
                device_id=(my_i,),
                device_id_type=pl.DeviceIdType.MESH,
            )
            wait.wait_send()

    return pl.pallas_call(
        body,
        out_shape=jax.ShapeDtypeStruct((b, d_out), jnp.float32),
        in_specs=[pl.BlockSpec(memory_space=pltpu.VMEM)] * 3,
        out_specs=pl.BlockSpec(memory_space=pltpu.VMEM),
        scratch_shapes=[
            pltpu.VMEM((b, h_dim), jnp.float32),
            pltpu.VMEM((N_DEV, b, chunk), jnp.float32),
            pltpu.VMEM((b, chunk), jnp.float32),
            pltpu.VMEM((b, h_dim), jnp.float32),
            pltpu.SemaphoreType.DMA,
            pltpu.SemaphoreType.DMA,
            pltpu.SemaphoreType.DMA,
            pltpu.SemaphoreType.DMA,
        ],
    )(x_shard, win, wout)


def kernel(x, Win0, Wout0, Win1, Wout1, Win2, Wout2):
    x = _mlp_layer(x, Win0, Wout0)
    x = _mlp_layer(x, Win1, Wout1)
    x = _mlp_layer(x, Win2, Wout2)
    return x


# device time: 76239 ns/iter; 1.9546x vs baseline; 1.1632x over previous
import jax
import jax.numpy as jnp
from jax import lax
from jax.experimental import pallas as pl
from jax.experimental.pallas import tpu as pltpu

N_DEV = 16


def kernel(x, Win0, Wout0, Win1, Wout1, Win2, Wout2):
    b, d_in = x.shape
    _, h_dim = Win0.shape
    _, d_out = Wout0.shape
    chunk = h_dim // N_DEV

    def body(x_ref, win0_ref, wout0_ref, win1_ref, wout1_ref,
             win2_ref, wout2_ref, out_ref,
             acc_ref, rs_recv_ref, hchunk_ref, hfull_ref, xbuf_ref,
             rs_send_sem, rs_recv_sem, ag_send_sem, ag_recv_sem):
        my_i = lax.axis_index("i")

        def layer(xin_ref, win_ref, wout_ref, xout_ref):
            acc_ref[:, :] = jnp.dot(
                xin_ref[:, :], win_ref[:, :],
                preferred_element_type=jnp.float32,
            )

            for k in range(N_DEV - 1):
                tgt = (my_i + 1 + k) % N_DEV
                rdma = pltpu.make_async_remote_copy(
                    src_ref=acc_ref.at[:, pl.ds(tgt * chunk, chunk)],
                    dst_ref=rs_recv_ref.at[my_i],
                    send_sem=rs_send_sem,
                    recv_sem=rs_recv_sem,
                    device_id=(tgt,),
                    device_id_type=pl.DeviceIdType.MESH,
                )
                rdma.start()

            rs_recv_ref[my_i] = acc_ref[:, pl.ds(my_i * chunk, chunk)]

            for k in range(N_DEV - 1):
                wait = pltpu.make_async_remote_copy(
                    src_ref=acc_ref.at[:, pl.ds(0, chunk)],
                    dst_ref=rs_recv_ref.at[k],
                    send_sem=rs_send_sem,
                    recv_sem=rs_recv_sem,
                    device_id=(my_i,),
                    device_id_type=pl.DeviceIdType.MESH,
                )
                wait.wait_recv()

            h_me = rs_recv_ref[0]
            for j in range(1, N_DEV):
                h_me = h_me + rs_recv_ref[j]
            hchunk_ref[:, :] = jnp.maximum(h_me, 0.0)

            for k in range(N_DEV - 1):
                wait = pltpu.make_async_remote_copy(
                    src_ref=acc_ref.at[:, pl.ds(0, chunk)],
                    dst_ref=rs_recv_ref.at[k],
                    send_sem=rs_send_sem,
                    recv_sem=rs_recv_sem,
                    device_id=(my_i,),
                    device_id_type=pl.DeviceIdType.MESH,
                )
                wait.wait_send()

            for k in range(N_DEV - 1):
                tgt = (my_i + 1 + k) % N_DEV
                rdma = pltpu.make_async_remote_copy(
                    src_ref=hchunk_ref,
                    dst_ref=hfull_ref.at[:, pl.ds(my_i * chunk, chunk)],
                    send_sem=ag_send_sem,
                    recv_sem=ag_recv_sem,
                    device_id=(tgt,),
                    device_id_type=pl.DeviceIdType.MESH,
                )
                rdma.start()

            hfull_ref[:, pl.ds(my_i * chunk, chunk)] = hchunk_ref[:, :]

            for k in range(N_DEV - 1):
                wait = pltpu.make_async_remote_copy(
                    src_ref=hchunk_ref,
                    dst_ref=hfull_ref.at[:, pl.ds(k * chunk, chunk)],
                    send_sem=ag_send_sem,
                    recv_sem=ag_recv_sem,
                    device_id=(my_i,),
                    device_id_type=pl.DeviceIdType.MESH,
                )
                wait.wait_recv()

            xout_ref[:, :] = jnp.dot(
                hfull_ref[:, :], wout_ref[:, :],
                preferred_element_type=jnp.float32,
            )

            for k in range(N_DEV - 1):
                wait = pltpu.make_async_remote_copy(
                    src_ref=hchunk_ref,
                    dst_ref=hfull_ref.at[:, pl.ds(k * chunk, chunk)],
                    send_sem=ag_send_sem,
                    recv_sem=ag_recv_sem,
                    device_id=(my_i,),
                    device_id_type=pl.DeviceIdType.MESH,
                )
                wait.wait_send()

        layer(x_ref, win0_ref, wout0_ref, xbuf_ref)
        layer(xbuf_ref, win1_ref, wout1_ref, xbuf_ref)
        layer(xbuf_ref, win2_ref, wout2_ref, out_ref)

    return pl.pallas_call(
        body,
        out_shape=jax.ShapeDtypeStruct((b, d_out), jnp.float32),
        in_specs=[pl.BlockSpec(memory_space=pltpu.VMEM)] * 7,
        out_specs=pl.BlockSpec(memory_space=pltpu.VMEM),
        scratch_shapes=[
            pltpu.VMEM((b, h_dim), jnp.float32),
            pltpu.VMEM((N_DEV, b, chunk), jnp.float32),
            pltpu.VMEM((b, chunk), jnp.float32),
            pltpu.VMEM((b, h_dim), jnp.float32),
            pltpu.VMEM((b, d_in), jnp.float32),
            pltpu.SemaphoreType.DMA,
            pltpu.SemaphoreType.DMA,
            pltpu.SemaphoreType.DMA,
            pltpu.SemaphoreType.DMA,
        ],
        compiler_params=pltpu.CompilerParams(
            vmem_limit_bytes=100 * 1024 * 1024,
        ),
    )(x, Win0, Wout0, Win1, Wout1, Win2, Wout2)
